# baseline (speedup 1.0000x reference)
import jax
import jax.numpy as jnp
from jax import lax
from jax.experimental import pallas as pl
from jax.experimental.pallas import tpu as pltpu
from jax.experimental.pallas import tpu_sc as plsc

DIM = 2048
SEQ_LEN = 8192
NUM_CORES = 2
NUM_SUBCORES = 16
LANES = 16
NUM_WORKERS = NUM_CORES * NUM_SUBCORES
ROWS_PER_WORKER = SEQ_LEN // NUM_WORKERS
SLOT_ROWS = 16
NUM_SLOTS = ROWS_PER_WORKER // SLOT_ROWS
NBUF = 3
VECS_PER_ROW = DIM // LANES
INNER_VECS = 16
BLOCKS_PER_ROW = VECS_PER_ROW // INNER_VECS


def _scale_slot(buf, scale):
    @plsc.parallel_loop(0, SLOT_ROWS * BLOCKS_PER_ROW)
    def _blk(v):
        row = v // BLOCKS_PER_ROW
        col0 = (v % BLOCKS_PER_ROW) * (INNER_VECS * LANES)
        for u in range(INNER_VECS):
            sl = pl.ds(col0 + u * LANES, LANES)
            buf[row, sl] = buf[row, sl] * scale


def _sc_body(emb_hbm, out_hbm, b0, b1, b2, i0, i1, i2, o0, o1, o2):
    scale = jnp.float32(DIM ** -0.5)
    bufs = (b0, b1, b2)
    isems = (i0, i1, i2)
    osems = (o0, o1, o2)
    wid = lax.axis_index("s") * NUM_CORES + lax.axis_index("c")
    base = wid * ROWS_PER_WORKER

    def in_slice(k):
        return emb_hbm.at[pl.ds(base + k * SLOT_ROWS, SLOT_ROWS)]

    def out_slice(k):
        return out_hbm.at[pl.ds(base + k * SLOT_ROWS, SLOT_ROWS)]

    def slot(k, b, reclaim, prefetch):
        pltpu.make_async_copy(in_slice(k), bufs[b], isems[b]).wait()
        _scale_slot(bufs[b], scale)
        pltpu.async_copy(bufs[b], out_slice(k), osems[b])
        if reclaim:
            bp = (b + 2) % NBUF
            pltpu.make_async_copy(bufs[bp], out_slice(k - 1), osems[bp]).wait()
            if prefetch:
                pltpu.async_copy(in_slice(k + 2), bufs[bp], isems[bp])

    for b in range(NBUF):
        pltpu.async_copy(in_slice(b), bufs[b], isems[b])
    slot(0, 0, False, False)
    slot(1, 1, True, True)
    slot(2, 2, True, True)

    @pl.loop(1, 4)
    def _group(g):
        for b in range(NBUF):
            k = NBUF * g + b
            slot(k, b, True, True)

    slot(12, 0, True, True)
    slot(13, 1, True, True)
    slot(14, 2, True, False)
    slot(15, 0, True, False)

    pltpu.make_async_copy(bufs[0], out_slice(15), osems[0]).wait()


_SCRATCH = (
    [pltpu.VMEM((SLOT_ROWS, DIM), jnp.float32)] * NBUF
    + [pltpu.SemaphoreType.DMA] * (2 * NBUF)
)

_pos_emb_sc = pl.kernel(
    _sc_body,
    out_type=jax.ShapeDtypeStruct((SEQ_LEN, DIM), jnp.float32),
    mesh=plsc.VectorSubcoreMesh(core_axis_name="c", subcore_axis_name="s"),
    scratch_types=_SCRATCH,
)


def kernel(x, emb):
    seq_len = x.shape[1]
    assert seq_len == SEQ_LEN
    return _pos_emb_sc(emb)


# speedup vs baseline: 1.0156x; 1.0156x over previous
import jax
import jax.numpy as jnp
from jax import lax
from jax.experimental import pallas as pl
from jax.experimental.pallas import tpu as pltpu
from jax.experimental.pallas import tpu_sc as plsc

DIM = 2048
SEQ_LEN = 8192
NUM_CORES = 2
NUM_SUBCORES = 16
LANES = 16
NUM_WORKERS = NUM_CORES * NUM_SUBCORES
ROWS_PER_WORKER = SEQ_LEN // NUM_WORKERS
SLOT_ROWS = 16
NUM_SLOTS = ROWS_PER_WORKER // SLOT_ROWS
HALF_ROWS = SLOT_ROWS // 2
VECS_PER_ROW = DIM // LANES


INNER_VECS = 16
BLOCKS_PER_ROW = VECS_PER_ROW // INNER_VECS


def _scale_half(src, src_row0, dst, scale):
    @plsc.parallel_loop(0, HALF_ROWS * BLOCKS_PER_ROW)
    def _blk(v):
        row = v // BLOCKS_PER_ROW
        col0 = (v % BLOCKS_PER_ROW) * (INNER_VECS * LANES)
        for u in range(INNER_VECS):
            sl = pl.ds(col0 + u * LANES, LANES)
            dst[row, sl] = src[src_row0 + row, sl] * scale


def _sc_body(emb_hbm, out_hbm, in0, in1, st0, st1, isem0, isem1, osem0, osem1):
    scale = jnp.float32(DIM ** -0.5)
    in_bufs = (in0, in1)
    in_sems = (isem0, isem1)
    out_bufs = (st0, st1)
    out_sems = (osem0, osem1)
    wid = lax.axis_index("s") * NUM_CORES + lax.axis_index("c")
    base = wid * ROWS_PER_WORKER

    def in_slice(k):
        return emb_hbm.at[pl.ds(base + k * SLOT_ROWS, SLOT_ROWS)]

    def in_half(k, h):
        return emb_hbm.at[pl.ds(base + k * SLOT_ROWS + h * HALF_ROWS, HALF_ROWS)]

    def issue_in(k, b):
        for h in range(2):
            pltpu.async_copy(
                in_half(k, h), in_bufs[b].at[pl.ds(h * HALF_ROWS, HALF_ROWS)],
                in_sems[b])

    def wait_in(k, b):
        for h in range(2):
            pltpu.make_async_copy(
                in_half(k, h), in_bufs[b].at[pl.ds(h * HALF_ROWS, HALF_ROWS)],
                in_sems[b]).wait()

    def out_half_slice(k, h):
        return out_hbm.at[pl.ds(base + k * SLOT_ROWS + h * HALF_ROWS, HALF_ROWS)]

    def slot(k, b, first):
        wait_in(k, b)
        for h in range(2):
            if not first:
                pltpu.make_async_copy(
                    out_bufs[h], out_half_slice(k - 1, h), out_sems[h]
                ).wait()
            _scale_half(in_bufs[b], h * HALF_ROWS, out_bufs[h], scale)
            pltpu.async_copy(out_bufs[h], out_half_slice(k, h), out_sems[h])

    issue_in(0, 0)
    issue_in(1, 1)
    slot(0, 0, True)
    issue_in(2, 0)
    slot(1, 1, False)
    issue_in(3, 1)

    @pl.loop(1, NUM_SLOTS // 2 - 1)
    def _group(g):
        for b in range(2):
            k = 2 * g + b
            slot(k, b, False)
            issue_in(k + 2, b)

    slot(NUM_SLOTS - 2, 0, False)
    slot(NUM_SLOTS - 1, 1, False)

    for h in range(2):
        pltpu.make_async_copy(
            out_bufs[h], out_half_slice(NUM_SLOTS - 1, h), out_sems[h]
        ).wait()


_SCRATCH = (
    [pltpu.VMEM((SLOT_ROWS, DIM), jnp.float32)] * 2
    + [pltpu.VMEM((HALF_ROWS, DIM), jnp.float32)] * 2
    + [pltpu.SemaphoreType.DMA] * 4
)

_pos_emb_sc = pl.kernel(
    _sc_body,
    out_type=jax.ShapeDtypeStruct((SEQ_LEN, DIM), jnp.float32),
    mesh=plsc.VectorSubcoreMesh(core_axis_name="c", subcore_axis_name="s"),
    scratch_types=_SCRATCH,
)


def kernel(x, emb):
    seq_len = x.shape[1]
    assert seq_len == SEQ_LEN
    return _pos_emb_sc(emb)
